# baseline (speedup 1.0000x reference)
import functools

import jax
import jax.numpy as jnp
from jax import lax
from jax.experimental import pallas as pl
from jax.experimental.pallas import tpu as pltpu
from jax.experimental.pallas import tpu_sc as plsc

_N = 10000
_F = 128

_NC = 2
_NS = 16
_NW = _NC * _NS

_CH = 128
_E2 = 327680
_EPT = _E2 // _NW
_HEPT = _EPT // 2
_NCHH = _HEPT // _CH
_EPT_DEG = _E2 // _NS
_NCH_DEG = _EPT_DEG // _CH
_DEG_PAD = 10240

_mesh = plsc.VectorSubcoreMesh(core_axis_name="c", subcore_axis_name="s")


def _newton_rsqrt(x):
    xc = jnp.maximum(x, 1e-12)
    i = lax.bitcast_convert_type(xc, jnp.int32)
    i = jnp.int32(0x5F3759DF) - (i >> 1)
    y = lax.bitcast_convert_type(i, jnp.float32)
    for _ in range(3):
        y = y * (1.5 - 0.5 * xc * y * y)
    return jnp.where(x > 0.0, y, 0.0)


@functools.partial(
    pl.kernel,
    out_type=jax.ShapeDtypeStruct((_E2,), jnp.float32),
    mesh=_mesh,
    scratch_types=[
        pltpu.VMEM((_EPT,), jnp.int32),
        pltpu.VMEM((_EPT_DEG,), jnp.int32),
        pltpu.VMEM((_EPT_DEG,), jnp.float32),
        pltpu.VMEM((_CH,), jnp.float32),
        pltpu.VMEM((_CH,), jnp.float32),
        pltpu.VMEM((640,), jnp.float32),
        pltpu.VMEM_SHARED((_DEG_PAD,), jnp.float32),
        pltpu.SemaphoreType.DMA,
        pltpu.SemaphoreType.DMA,
    ],
)
def _prep(src_hbm, dst_hbm, ew_hbm, norm_hbm,
          svb, dvb, ewb, gs, gd, db, deg_sp, sem, sem2):
    s = lax.axis_index("s")
    c = lax.axis_index("c")
    zeros = jnp.zeros((16,), jnp.float32)

    def _z(g, carry):
        db[pl.ds(g * 16, 16)] = zeros
        return carry
    lax.fori_loop(0, 40, _z, 0)
    pltpu.sync_copy(db, deg_sp.at[pl.ds(s * 640, 640)])
    plsc.subcore_barrier()

    d0 = s * _EPT_DEG
    pltpu.sync_copy(dst_hbm.at[pl.ds(d0, _EPT_DEG)], dvb)
    pltpu.sync_copy(ew_hbm.at[pl.ds(d0, _EPT_DEG)], ewb)

    def _deg(j, carry):
        sl = pl.ds(j * _CH, _CH)
        pltpu.sync_copy(ewb.at[sl], deg_sp.at[dvb.at[sl]], add=True)
        return carry
    lax.fori_loop(0, _NCH_DEG, _deg, 0)
    plsc.subcore_barrier()

    pltpu.sync_copy(deg_sp.at[pl.ds(s * 640, 640)], db)

    def _rs(g, carry):
        xv = db[pl.ds(g * 16, 16)]
        db[pl.ds(g * 16, 16)] = _newton_rsqrt(xv)
        return carry
    lax.fori_loop(0, 40, _rs, 0)
    pltpu.sync_copy(db, deg_sp.at[pl.ds(s * 640, 640)])
    plsc.subcore_barrier()

    wid = c * _NS + s
    b0 = wid * _EPT
    pltpu.sync_copy(src_hbm.at[pl.ds(b0, _EPT)], svb)
    pltpu.sync_copy(dst_hbm.at[pl.ds(b0, _EPT)], dvb.at[pl.ds(0, _EPT)])
    pltpu.sync_copy(ew_hbm.at[pl.ds(b0, _EPT)], ewb.at[pl.ds(0, _EPT)])

    def _nrm(j, carry):
        sl = pl.ds(j * _CH, _CH)
        pltpu.async_copy(deg_sp.at[svb.at[sl]], gs, sem).wait()
        pltpu.async_copy(deg_sp.at[dvb.at[sl]], gd, sem2).wait()
        for g in range(_CH // 16):
            s16 = pl.ds(g * 16, 16)
            e16 = pl.ds(j * _CH + g * 16, 16)
            ewb[e16] = gs[s16] * ewb[e16] * gd[s16]
        return carry
    lax.fori_loop(0, _EPT // _CH, _nrm, 0)
    pltpu.sync_copy(ewb.at[pl.ds(0, _EPT)], norm_hbm.at[pl.ds(b0, _EPT)])


@functools.partial(
    pl.kernel,
    out_type=jax.ShapeDtypeStruct((_NC, _N, _F), jnp.float32),
    mesh=_mesh,
    compiler_params=pltpu.CompilerParams(use_tc_tiling_on_sc=False),
    scratch_types=[
        pltpu.VMEM((_HEPT,), jnp.int32),
        pltpu.VMEM((_HEPT,), jnp.int32),
        pltpu.VMEM((_HEPT,), jnp.float32),
        pltpu.VMEM((_CH, _F // 2), jnp.int32),
        pltpu.VMEM((_CH, _F // 2), jnp.int32),
        pltpu.VMEM((_CH, _F), jnp.float32),
        pltpu.VMEM_SHARED((_N, _F), jnp.float32),
        pltpu.SemaphoreType.DMA,
        pltpu.SemaphoreType.DMA,
    ],
)
def _agg(m_hbm, src_hbm, dst_hbm, norm_hbm, p_hbm,
         svb, dvb, nvb, r0b, r1b, fbuf, acc_sp, g0, g1):
    s = lax.axis_index("s")
    c = lax.axis_index("c")
    zeros = jnp.zeros((16,), jnp.float32)
    rows = (r0b, r1b)
    gsem = (g0, g1)

    def _z(r, carry):
        for j in range(_F // 16):
            fbuf[r, pl.ds(j * 16, 16)] = zeros
        return carry
    lax.fori_loop(0, _CH, _z, 0)
    for t in range(5):
        @pl.when((s < _NS - 1) | (t < 3))
        def _():
            pltpu.sync_copy(fbuf, acc_sp.at[pl.ds(s * 640 + t * _CH, _CH)])

    @pl.when(s == _NS - 1)
    def _():
        pltpu.sync_copy(fbuf.at[pl.ds(0, 16)], acc_sp.at[pl.ds(9984, 16)])
    plsc.subcore_barrier()

    wid = c * _NS + s
    b0 = wid * _EPT

    def _start_gather(i, j):
        pltpu.async_copy(m_hbm.at[svb.at[pl.ds(j * _CH, _CH)]], rows[i],
                         gsem[i])

    def _wait_gather(i, j):
        pltpu.make_async_copy(m_hbm.at[svb.at[pl.ds(j * _CH, _CH)]], rows[i],
                              gsem[i]).wait()

    def _scale(i, j):
        rbuf = rows[i]

        def _grp(g, carry):
            nv16 = nvb[pl.ds(j * _CH + g * 16, 16)]
            for e in range(16):
                w = jnp.full((16,), nv16[e], jnp.float32)
                for q in range(_F // 32):
                    xi = rbuf[g * 16 + e, pl.ds(q * 16, 16)]
                    a = lax.bitcast_convert_type(xi << 16, jnp.float32)
                    b2 = lax.bitcast_convert_type(
                        xi & jnp.int32(-65536), jnp.float32)
                    fbuf[g * 16 + e, pl.ds(q * 32, 16)] = a * w
                    fbuf[g * 16 + e, pl.ds(q * 32 + 16, 16)] = b2 * w
            return carry
        lax.fori_loop(0, _CH // 16, _grp, 0)

    for h in range(2):
        e0 = b0 + h * _HEPT
        pltpu.sync_copy(src_hbm.at[pl.ds(e0, _HEPT)], svb)
        pltpu.sync_copy(dst_hbm.at[pl.ds(e0, _HEPT)], dvb)
        pltpu.sync_copy(norm_hbm.at[pl.ds(e0, _HEPT)], nvb)
        _start_gather(0, 0)
        _start_gather(1, 1)

        def _outer(b, carry):
            for i in range(2):
                j = b * 2 + i
                _wait_gather(i, j)
                _scale(i, j)
                pltpu.sync_copy(fbuf,
                                acc_sp.at[dvb.at[pl.ds(j * _CH, _CH)]],
                                add=True)

                @pl.when(j + 2 < _NCHH)
                def _():
                    _start_gather(i, j + 2)
            return carry
        lax.fori_loop(0, _NCHH // 2, _outer, 0)
    plsc.subcore_barrier()

    for t in range(5):
        q0 = s * 640 + t * _CH

        @pl.when((s < _NS - 1) | (t < 3))
        def _():
            pltpu.sync_copy(acc_sp.at[pl.ds(q0, _CH)],
                            p_hbm.at[c, pl.ds(q0, _CH)])

    @pl.when(s == _NS - 1)
    def _():
        pltpu.sync_copy(acc_sp.at[pl.ds(9984, 16)], p_hbm.at[c, pl.ds(9984, 16)])


_BLK = 2000


def _pack_perm(h):
    n = h.shape[0]
    h4 = h.reshape(n, 4, 2, 16)
    ai = lax.bitcast_convert_type(h4[:, :, 0, :].reshape(n, _F // 2), jnp.int32)
    bi = lax.bitcast_convert_type(h4[:, :, 1, :].reshape(n, _F // 2), jnp.int32)

    def _r16(x):
        return (x + 0x7FFF + ((x >> 16) & 1)) >> 16

    return (_r16(ai) & 0xFFFF) | (_r16(bi) << 16)


def _mm_body(x_ref, w_ref, o_ref):
    o_ref[...] = _pack_perm(
        jnp.dot(x_ref[...], w_ref[...], preferred_element_type=jnp.float32))


def _matmul(x, w):
    return pl.pallas_call(
        _mm_body,
        grid=(_N // _BLK,),
        in_specs=[
            pl.BlockSpec((_BLK, _F), lambda i: (i, 0)),
            pl.BlockSpec((_F, _F), lambda i: (0, 0)),
        ],
        out_specs=pl.BlockSpec((_BLK, _F // 2), lambda i: (i, 0)),
        out_shape=jax.ShapeDtypeStruct((_N, _F // 2), jnp.int32),
    )(x, w)


def _ep1_body(p_ref, b_ref, w_ref, xin_ref, m2_ref):
    xin = p_ref[0] + p_ref[1] + b_ref[...]
    h = xin + jnp.maximum(xin, 0.0)
    xin_ref[...] = xin
    m2_ref[...] = _pack_perm(
        jnp.dot(h, w_ref[...], preferred_element_type=jnp.float32))


def _ep1(p, b, w):
    return pl.pallas_call(
        _ep1_body,
        grid=(_N // _BLK,),
        in_specs=[
            pl.BlockSpec((_NC, _BLK, _F), lambda i: (0, i, 0)),
            pl.BlockSpec((1, _F), lambda i: (0, 0)),
            pl.BlockSpec((_F, _F), lambda i: (0, 0)),
        ],
        out_specs=[
            pl.BlockSpec((_BLK, _F), lambda i: (i, 0)),
            pl.BlockSpec((_BLK, _F // 2), lambda i: (i, 0)),
        ],
        out_shape=[
            jax.ShapeDtypeStruct((_N, _F), jnp.float32),
            jax.ShapeDtypeStruct((_N, _F // 2), jnp.int32),
        ],
    )(p, b, w)


def _ep2_body(p_ref, b_ref, xin_ref, w_ref, m3_ref):
    c1 = p_ref[0] + p_ref[1] + b_ref[...]
    h2 = xin_ref[...] + jnp.maximum(c1, 0.0)
    m3_ref[...] = _pack_perm(
        jnp.dot(h2, w_ref[...], preferred_element_type=jnp.float32))


def _ep2(p, b, xin, w):
    return pl.pallas_call(
        _ep2_body,
        grid=(_N // _BLK,),
        in_specs=[
            pl.BlockSpec((_NC, _BLK, _F), lambda i: (0, i, 0)),
            pl.BlockSpec((1, _F), lambda i: (0, 0)),
            pl.BlockSpec((_BLK, _F), lambda i: (i, 0)),
            pl.BlockSpec((_F, _F), lambda i: (0, 0)),
        ],
        out_specs=pl.BlockSpec((_BLK, _F // 2), lambda i: (i, 0)),
        out_shape=jax.ShapeDtypeStruct((_N, _F // 2), jnp.int32),
    )(p, b, xin, w)


def _ep3_body(p_ref, b_ref, o_ref):
    o_ref[...] = p_ref[0] + p_ref[1] + b_ref[...]


def _ep3(p, b):
    return pl.pallas_call(
        _ep3_body,
        grid=(_N // _BLK,),
        in_specs=[
            pl.BlockSpec((_NC, _BLK, _F), lambda i: (0, i, 0)),
            pl.BlockSpec((1, _F), lambda i: (0, 0)),
        ],
        out_specs=pl.BlockSpec((_BLK, _F), lambda i: (i, 0)),
        out_shape=jax.ShapeDtypeStruct((_N, _F), jnp.float32),
    )(p, b)


def kernel(x, edge_index, edge_attr, W1, b1, W2, b2, W3, b3):
    src = edge_index[0]
    dst = edge_index[1]
    npad = _E2 - src.shape[0]
    fill = jnp.arange(npad, dtype=jnp.int32) % _N
    src2 = jnp.concatenate([src, fill])
    dst2 = jnp.concatenate([dst, fill])
    ew2 = jnp.concatenate([edge_attr, jnp.zeros((npad,), jnp.float32)])

    norm = _prep(src2, dst2, ew2)

    m1 = _matmul(x, W1)
    p1 = _agg(m1, src2, dst2, norm)
    xin, m2 = _ep1(p1, b1.reshape(1, _F), W2)
    p2 = _agg(m2, src2, dst2, norm)
    m3 = _ep2(p2, b2.reshape(1, _F), xin, W3)
    p3 = _agg(m3, src2, dst2, norm)
    out = _ep3(p3, b3.reshape(1, _F))
    return out


# speedup vs baseline: 2.1466x; 2.1466x over previous
import functools

import jax
import jax.numpy as jnp
from jax import lax
from jax.experimental import pallas as pl
from jax.experimental.pallas import tpu as pltpu
from jax.experimental.pallas import tpu_sc as plsc

_N = 10000
_F = 128

_NC = 2
_NS = 16
_NW = _NC * _NS

_CH = 128
_E2 = 327680
_EPT = _E2 // _NW
_HEPT = _EPT // 2
_NCHH = _HEPT // _CH
_EPT_DEG = _E2 // _NS
_NCH_DEG = _EPT_DEG // _CH
_DEG_PAD = 10240

_mesh = plsc.VectorSubcoreMesh(core_axis_name="c", subcore_axis_name="s")


def _newton_rsqrt(x):
    xc = jnp.maximum(x, 1e-12)
    i = lax.bitcast_convert_type(xc, jnp.int32)
    i = jnp.int32(0x5F3759DF) - (i >> 1)
    y = lax.bitcast_convert_type(i, jnp.float32)
    for _ in range(3):
        y = y * (1.5 - 0.5 * xc * y * y)
    return jnp.where(x > 0.0, y, 0.0)


@functools.partial(
    pl.kernel,
    out_type=jax.ShapeDtypeStruct((_E2,), jnp.float32),
    mesh=_mesh,
    scratch_types=[
        pltpu.VMEM((_EPT,), jnp.int32),
        pltpu.VMEM((_EPT_DEG,), jnp.int32),
        pltpu.VMEM((_EPT_DEG,), jnp.float32),
        pltpu.VMEM((_CH,), jnp.float32),
        pltpu.VMEM((_CH,), jnp.float32),
        pltpu.VMEM((640,), jnp.float32),
        pltpu.VMEM_SHARED((_DEG_PAD,), jnp.float32),
        pltpu.SemaphoreType.DMA,
        pltpu.SemaphoreType.DMA,
    ],
)
def _prep(src_hbm, dst_hbm, ew_hbm, norm_hbm,
          svb, dvb, ewb, gs, gd, db, deg_sp, sem, sem2):
    s = lax.axis_index("s")
    c = lax.axis_index("c")
    zeros = jnp.zeros((16,), jnp.float32)

    def _z(g, carry):
        db[pl.ds(g * 16, 16)] = zeros
        return carry
    lax.fori_loop(0, 40, _z, 0)
    pltpu.sync_copy(db, deg_sp.at[pl.ds(s * 640, 640)])
    plsc.subcore_barrier()

    d0 = s * _EPT_DEG
    pltpu.sync_copy(dst_hbm.at[pl.ds(d0, _EPT_DEG)], dvb)
    pltpu.sync_copy(ew_hbm.at[pl.ds(d0, _EPT_DEG)], ewb)

    def _deg(j, carry):
        sl = pl.ds(j * _CH, _CH)
        pltpu.sync_copy(ewb.at[sl], deg_sp.at[dvb.at[sl]], add=True)
        return carry
    lax.fori_loop(0, _NCH_DEG, _deg, 0)
    plsc.subcore_barrier()

    pltpu.sync_copy(deg_sp.at[pl.ds(s * 640, 640)], db)

    def _rs(g, carry):
        xv = db[pl.ds(g * 16, 16)]
        db[pl.ds(g * 16, 16)] = _newton_rsqrt(xv)
        return carry
    lax.fori_loop(0, 40, _rs, 0)
    pltpu.sync_copy(db, deg_sp.at[pl.ds(s * 640, 640)])
    plsc.subcore_barrier()

    wid = c * _NS + s
    b0 = wid * _EPT
    pltpu.sync_copy(src_hbm.at[pl.ds(b0, _EPT)], svb)
    pltpu.sync_copy(dst_hbm.at[pl.ds(b0, _EPT)], dvb.at[pl.ds(0, _EPT)])
    pltpu.sync_copy(ew_hbm.at[pl.ds(b0, _EPT)], ewb.at[pl.ds(0, _EPT)])

    def _nrm(j, carry):
        sl = pl.ds(j * _CH, _CH)
        pltpu.async_copy(deg_sp.at[svb.at[sl]], gs, sem).wait()
        pltpu.async_copy(deg_sp.at[dvb.at[sl]], gd, sem2).wait()
        for g in range(_CH // 16):
            s16 = pl.ds(g * 16, 16)
            e16 = pl.ds(j * _CH + g * 16, 16)
            ewb[e16] = gs[s16] * ewb[e16] * gd[s16]
        return carry
    lax.fori_loop(0, _EPT // _CH, _nrm, 0)
    pltpu.sync_copy(ewb.at[pl.ds(0, _EPT)], norm_hbm.at[pl.ds(b0, _EPT)])


@functools.partial(
    pl.kernel,
    out_type=jax.ShapeDtypeStruct((_NC, _N, _F), jnp.float32),
    mesh=_mesh,
    scratch_types=[
        pltpu.VMEM((_HEPT,), jnp.int32),
        pltpu.VMEM((_HEPT,), jnp.int32),
        pltpu.VMEM((_HEPT,), jnp.float32),
        pltpu.VMEM((_CH, _F), jnp.float32),
        pltpu.VMEM((_CH, _F), jnp.float32),
        pltpu.VMEM_SHARED((_N, _F), jnp.float32),
        pltpu.SemaphoreType.DMA,
        pltpu.SemaphoreType.DMA,
    ],
)
def _agg(m_hbm, src_hbm, dst_hbm, norm_hbm, p_hbm,
         svb, dvb, nvb, r0b, r1b, acc_sp, g0, g1):
    s = lax.axis_index("s")
    c = lax.axis_index("c")
    zeros = jnp.zeros((16,), jnp.float32)
    rows = (r0b, r1b)
    gsem = (g0, g1)

    def _z(r, carry):
        for j in range(_F // 16):
            r0b[r, pl.ds(j * 16, 16)] = zeros
        return carry
    lax.fori_loop(0, _CH, _z, 0)
    for t in range(5):
        @pl.when((s < _NS - 1) | (t < 3))
        def _():
            pltpu.sync_copy(r0b, acc_sp.at[pl.ds(s * 640 + t * _CH, _CH)])

    @pl.when(s == _NS - 1)
    def _():
        pltpu.sync_copy(r0b.at[pl.ds(0, 16)], acc_sp.at[pl.ds(9984, 16)])
    plsc.subcore_barrier()

    wid = c * _NS + s
    b0 = wid * _EPT

    def _start_gather(i, j):
        pltpu.async_copy(m_hbm.at[svb.at[pl.ds(j * _CH, _CH)]], rows[i],
                         gsem[i])

    def _wait_gather(i, j):
        pltpu.make_async_copy(m_hbm.at[svb.at[pl.ds(j * _CH, _CH)]], rows[i],
                              gsem[i]).wait()

    def _scale(i, j):
        rbuf = rows[i]

        def _grp(g, carry):
            nv16 = nvb[pl.ds(j * _CH + g * 16, 16)]
            for e in range(16):
                w = jnp.full((16,), nv16[e], jnp.float32)
                for q in range(_F // 16):
                    sl = pl.ds(q * 16, 16)
                    rbuf[g * 16 + e, sl] = rbuf[g * 16 + e, sl] * w
            return carry
        lax.fori_loop(0, _CH // 16, _grp, 0)

    for h in range(2):
        e0 = b0 + h * _HEPT
        pltpu.sync_copy(src_hbm.at[pl.ds(e0, _HEPT)], svb)
        pltpu.sync_copy(dst_hbm.at[pl.ds(e0, _HEPT)], dvb)
        pltpu.sync_copy(norm_hbm.at[pl.ds(e0, _HEPT)], nvb)
        _start_gather(0, 0)
        _start_gather(1, 1)

        def _outer(b, carry):
            for i in range(2):
                j = b * 2 + i
                _wait_gather(i, j)
                _scale(i, j)
                pltpu.sync_copy(rows[i],
                                acc_sp.at[dvb.at[pl.ds(j * _CH, _CH)]],
                                add=True)

                @pl.when(j + 2 < _NCHH)
                def _():
                    _start_gather(i, j + 2)
            return carry
        lax.fori_loop(0, _NCHH // 2, _outer, 0)
    plsc.subcore_barrier()

    for t in range(5):
        q0 = s * 640 + t * _CH

        @pl.when((s < _NS - 1) | (t < 3))
        def _():
            pltpu.sync_copy(acc_sp.at[pl.ds(q0, _CH)],
                            p_hbm.at[c, pl.ds(q0, _CH)])

    @pl.when(s == _NS - 1)
    def _():
        pltpu.sync_copy(acc_sp.at[pl.ds(9984, 16)], p_hbm.at[c, pl.ds(9984, 16)])


_BLK = 2000


def _mm_body(x_ref, w_ref, o_ref):
    o_ref[...] = jnp.dot(x_ref[...], w_ref[...], preferred_element_type=jnp.float32)


def _matmul(x, w):
    return pl.pallas_call(
        _mm_body,
        grid=(_N // _BLK,),
        in_specs=[
            pl.BlockSpec((_BLK, _F), lambda i: (i, 0)),
            pl.BlockSpec((_F, _F), lambda i: (0, 0)),
        ],
        out_specs=pl.BlockSpec((_BLK, _F), lambda i: (i, 0)),
        out_shape=jax.ShapeDtypeStruct((_N, _F), jnp.float32),
    )(x, w)


def _ep1_body(p_ref, b_ref, w_ref, xin_ref, m2_ref):
    xin = p_ref[0] + p_ref[1] + b_ref[...]
    h = xin + jnp.maximum(xin, 0.0)
    xin_ref[...] = xin
    m2_ref[...] = jnp.dot(h, w_ref[...], preferred_element_type=jnp.float32)


def _ep1(p, b, w):
    return pl.pallas_call(
        _ep1_body,
        grid=(_N // _BLK,),
        in_specs=[
            pl.BlockSpec((_NC, _BLK, _F), lambda i: (0, i, 0)),
            pl.BlockSpec((1, _F), lambda i: (0, 0)),
            pl.BlockSpec((_F, _F), lambda i: (0, 0)),
        ],
        out_specs=[
            pl.BlockSpec((_BLK, _F), lambda i: (i, 0)),
            pl.BlockSpec((_BLK, _F), lambda i: (i, 0)),
        ],
        out_shape=[
            jax.ShapeDtypeStruct((_N, _F), jnp.float32),
            jax.ShapeDtypeStruct((_N, _F), jnp.float32),
        ],
    )(p, b, w)


def _ep2_body(p_ref, b_ref, xin_ref, w_ref, m3_ref):
    c1 = p_ref[0] + p_ref[1] + b_ref[...]
    h2 = xin_ref[...] + jnp.maximum(c1, 0.0)
    m3_ref[...] = jnp.dot(h2, w_ref[...], preferred_element_type=jnp.float32)


def _ep2(p, b, xin, w):
    return pl.pallas_call(
        _ep2_body,
        grid=(_N // _BLK,),
        in_specs=[
            pl.BlockSpec((_NC, _BLK, _F), lambda i: (0, i, 0)),
            pl.BlockSpec((1, _F), lambda i: (0, 0)),
            pl.BlockSpec((_BLK, _F), lambda i: (i, 0)),
            pl.BlockSpec((_F, _F), lambda i: (0, 0)),
        ],
        out_specs=pl.BlockSpec((_BLK, _F), lambda i: (i, 0)),
        out_shape=jax.ShapeDtypeStruct((_N, _F), jnp.float32),
    )(p, b, xin, w)


def _ep3_body(p_ref, b_ref, o_ref):
    o_ref[...] = p_ref[0] + p_ref[1] + b_ref[...]


def _ep3(p, b):
    return pl.pallas_call(
        _ep3_body,
        grid=(_N // _BLK,),
        in_specs=[
            pl.BlockSpec((_NC, _BLK, _F), lambda i: (0, i, 0)),
            pl.BlockSpec((1, _F), lambda i: (0, 0)),
        ],
        out_specs=pl.BlockSpec((_BLK, _F), lambda i: (i, 0)),
        out_shape=jax.ShapeDtypeStruct((_N, _F), jnp.float32),
    )(p, b)


def kernel(x, edge_index, edge_attr, W1, b1, W2, b2, W3, b3):
    src = edge_index[0]
    dst = edge_index[1]
    npad = _E2 - src.shape[0]
    fill = jnp.arange(npad, dtype=jnp.int32) % _N
    src2 = jnp.concatenate([src, fill])
    dst2 = jnp.concatenate([dst, fill])
    ew2 = jnp.concatenate([edge_attr, jnp.zeros((npad,), jnp.float32)])

    norm = _prep(src2, dst2, ew2)

    m1 = _matmul(x, W1)
    p1 = _agg(m1, src2, dst2, norm)
    xin, m2 = _ep1(p1, b1.reshape(1, _F), W2)
    p2 = _agg(m2, src2, dst2, norm)
    m3 = _ep2(p2, b2.reshape(1, _F), xin, W3)
    p3 = _agg(m3, src2, dst2, norm)
    out = _ep3(p3, b3.reshape(1, _F))
    return out
